# baseline (speedup 1.0000x reference)
import functools

import jax
import jax.numpy as jnp
from jax.experimental import pallas as pl
from jax.experimental.pallas import tpu as pltpu
from jax.experimental.pallas import tpu_sc as plsc

B, S, H = 1, 2048, 1024
E, K, FF = 8, 2, 2048
T = B * S
P = K * T

BLK = 256
NTILES = P // BLK + E
NROWS = NTILES * BLK
MAXT = T // BLK
NROWSD = NROWS + BLK

CT = 512
HP = H // 2

NC, NS = 2, 16
NW = NC * NS
RPW = P // NW
CH = 32
NCH = RPW // CH


def _router_kernel(x_ref, wgt_ref, dst_ref, gate_ref, tm_ref, loss_ref,
                   xp_ref):
    x = x_ref[...]
    wgt = wgt_ref[...]
    xp_ref[...] = x.astype(jnp.bfloat16)
    logits = jnp.dot(x, wgt, preferred_element_type=jnp.float32)
    iota_e = jax.lax.broadcasted_iota(jnp.int32, (T, E), 1)
    m1 = jnp.max(logits, axis=1, keepdims=True)
    a1 = jnp.min(jnp.where(logits == m1, iota_e, E), axis=1,
                 keepdims=True)
    masked = jnp.where(iota_e == a1, -jnp.inf, logits)
    m2 = jnp.max(masked, axis=1, keepdims=True)
    a2 = jnp.min(jnp.where(masked == m2, iota_e, E), axis=1,
                 keepdims=True)
    w0 = jax.nn.sigmoid(m1 - m2)
    gate_ref[:, 0:1] = w0
    gate_ref[:, 1:2] = 1.0 - w0

    a_pair = jnp.concatenate([a1, a2], axis=0)
    iota_pe = jax.lax.broadcasted_iota(jnp.int32, (P, E), 1)
    oh = (iota_pe == a_pair).astype(jnp.float32)

    c = oh
    s = 1
    while s < P:
        c = c + jnp.concatenate(
            [jnp.zeros((s, E), jnp.float32), c[:P - s, :]], axis=0)
        s *= 2
    ranks = c - oh

    counts = jnp.sum(oh, axis=0, keepdims=True)
    tiles_e = jnp.floor((counts + (BLK - 1)) * (1.0 / BLK))
    padded = tiles_e * BLK
    incl = padded
    cumt = tiles_e
    for s in (1, 2, 4):
        z = jnp.zeros((1, s), jnp.float32)
        incl = incl + jnp.concatenate([z, incl[:, :E - s]], axis=1)
        cumt = cumt + jnp.concatenate([z, cumt[:, :E - s]], axis=1)
    off = incl - padded

    rank_sel = jnp.sum(oh * ranks, axis=1, keepdims=True)
    off_sel = jnp.sum(oh * off, axis=1, keepdims=True)
    dst_ref[...] = (rank_sel + off_sel).astype(jnp.int32)

    iota_t = jax.lax.broadcasted_iota(jnp.int32, (NTILES, E), 0)
    tm_ref[...] = jnp.sum((iota_t >= cumt).astype(jnp.int32), axis=1,
                          keepdims=True)

    load = counts * (1.0 / T)
    loss_ref[...] = (0.01 * jnp.sum((load - 1.0 / E) ** 2)).reshape(1, 1)


def _router(x_flat, wgt):
    return pl.pallas_call(
        _router_kernel,
        out_shape=(
            jax.ShapeDtypeStruct((P, 1), jnp.int32),
            jax.ShapeDtypeStruct((T, K), jnp.float32),
            jax.ShapeDtypeStruct((NTILES, 1), jnp.int32),
            jax.ShapeDtypeStruct((1, 1), jnp.float32),
            jax.ShapeDtypeStruct((T, H), jnp.bfloat16),
        ),
    )(x_flat, wgt)


def _vmesh():
    return plsc.VectorSubcoreMesh(core_axis_name="core",
                                  subcore_axis_name="subcore")


def _sc_scatter(x_flat, dst_blk):
    @functools.partial(
        pl.kernel, mesh=_vmesh(),
        out_type=jax.ShapeDtypeStruct((NROWS, HP), jnp.float32),
        scratch_types=[
            pltpu.VMEM((NCH, CH), jnp.int32),
            pltpu.VMEM((CH, HP), jnp.float32),
            pltpu.VMEM((CH, HP), jnp.float32),
            pltpu.SemaphoreType.DMA,
            pltpu.SemaphoreType.DMA,
            pltpu.SemaphoreType.DMA,
            pltpu.SemaphoreType.DMA,
        ])
    def scatter_kernel(x_hbm, i_hbm, o_hbm, idx_v, ra, rb, sxa, sxb, ssa, ssb):
        wid = (jax.lax.axis_index("subcore") * NC
               + jax.lax.axis_index("core"))
        pltpu.sync_copy(i_hbm.at[pl.ds(wid * NCH, NCH)], idx_v)

        def src(q):
            return jax.lax.rem(wid * RPW + q * CH, T)

        bufs = (ra, rb)
        xsem = (sxa, sxb)
        ssem = (ssa, ssb)
        loads = {}
        stores = {}
        for q in (0, 1):
            loads[q] = pltpu.async_copy(
                x_hbm.at[pl.ds(src(q), CH)], bufs[q % 2], xsem[q % 2])
        for q in range(NCH):
            loads[q].wait()
            stores[q] = pltpu.async_copy(
                bufs[q % 2], o_hbm.at[idx_v.at[q]], ssem[q % 2])
            nq = q + 2
            if nq < NCH:
                stores[q].wait()
                loads[nq] = pltpu.async_copy(
                    x_hbm.at[pl.ds(src(nq), CH)], bufs[nq % 2], xsem[nq % 2])
        stores[NCH - 2].wait()
        stores[NCH - 1].wait()

    return scatter_kernel(x_flat, dst_blk)


def _sc_gather(ys, dst_blk):
    @functools.partial(
        pl.kernel, mesh=_vmesh(),
        out_type=jax.ShapeDtypeStruct((P, HP), jnp.float32),
        scratch_types=[
            pltpu.VMEM((NCH, CH), jnp.int32),
            pltpu.VMEM((CH, HP), jnp.float32),
            pltpu.VMEM((CH, HP), jnp.float32),
            pltpu.SemaphoreType.DMA,
            pltpu.SemaphoreType.DMA,
            pltpu.SemaphoreType.DMA,
            pltpu.SemaphoreType.DMA,
        ])
    def gather_kernel(ys_hbm, i_hbm, o_hbm, idx_v, ra, rb, sga, sgb, swa, swb):
        wid = (jax.lax.axis_index("subcore") * NC
               + jax.lax.axis_index("core"))
        base = wid * RPW
        pltpu.sync_copy(i_hbm.at[pl.ds(wid * NCH, NCH)], idx_v)

        bufs = (ra, rb)
        gsem = (sga, sgb)
        wsem = (swa, swb)
        gets = {}
        puts = {}
        for q in (0, 1):
            gets[q] = pltpu.async_copy(
                ys_hbm.at[idx_v.at[q]], bufs[q % 2], gsem[q % 2])
        for q in range(NCH):
            gets[q].wait()
            puts[q] = pltpu.async_copy(
                bufs[q % 2], o_hbm.at[pl.ds(base + q * CH, CH)], wsem[q % 2])
            nq = q + 2
            if nq < NCH:
                puts[q].wait()
                gets[nq] = pltpu.async_copy(
                    ys_hbm.at[idx_v.at[nq]], bufs[nq % 2], gsem[nq % 2])
        puts[NCH - 2].wait()
        puts[NCH - 1].wait()

    return gather_kernel(ys, dst_blk)


def _ffn_kernel(te_ref, xs_ref, w1_ref, b1_ref, w2_ref, b2_ref, ys_ref):
    j = pl.program_id(0)

    @pl.when(te_ref[j] < E)
    def _():
        x = xs_ref[...]
        h = jnp.dot(x, w1_ref[0].astype(jnp.bfloat16),
                    preferred_element_type=jnp.float32) + b1_ref[0]
        h = jax.nn.gelu(h)
        y = jnp.dot(h.astype(jnp.bfloat16), w2_ref[0].astype(jnp.bfloat16),
                    preferred_element_type=jnp.float32) + b2_ref[0]
        ys_ref[...] = y.astype(jnp.bfloat16)


def _ffn_grouped(te, xs, W1, b1, W2, b2):
    def emap(j, te_ref):
        return (jnp.minimum(te_ref[j], E - 1),)

    grid_spec = pltpu.PrefetchScalarGridSpec(
        num_scalar_prefetch=1,
        grid=(NTILES,),
        in_specs=[
            pl.BlockSpec((BLK, H), lambda j, te_ref: (j, 0)),
            pl.BlockSpec((1, H, FF), lambda j, te_ref: (*emap(j, te_ref), 0, 0)),
            pl.BlockSpec((1, 1, FF), lambda j, te_ref: (*emap(j, te_ref), 0, 0)),
            pl.BlockSpec((1, FF, H), lambda j, te_ref: (*emap(j, te_ref), 0, 0)),
            pl.BlockSpec((1, 1, H), lambda j, te_ref: (*emap(j, te_ref), 0, 0)),
        ],
        out_specs=pl.BlockSpec((BLK, H), lambda j, te_ref: (j, 0)),
    )
    return pl.pallas_call(
        _ffn_kernel,
        grid_spec=grid_spec,
        out_shape=jax.ShapeDtypeStruct((NROWS, H), jnp.bfloat16),
        compiler_params=pltpu.CompilerParams(
            dimension_semantics=("arbitrary",),
        ),
    )(te, xs, W1, b1.reshape(E, 1, FF), W2, b2.reshape(E, 1, H))


def _combine_kernel(gate_ref, g0_ref, g1_ref, out_ref):
    w0 = gate_ref[:, 0:1]
    w1 = gate_ref[:, 1:2]
    g0 = g0_ref[...].astype(jnp.float32)
    g1 = g1_ref[...].astype(jnp.float32)
    out_ref[...] = w0 * g0 + w1 * g1


def _combine(gate, g):
    return pl.pallas_call(
        _combine_kernel,
        grid=(T // CT,),
        in_specs=[
            pl.BlockSpec((CT, K), lambda c: (c, 0)),
            pl.BlockSpec((CT, H), lambda c: (c, 0)),
            pl.BlockSpec((CT, H), lambda c: (T // CT + c, 0)),
        ],
        out_specs=pl.BlockSpec((CT, H), lambda c: (c, 0)),
        out_shape=jax.ShapeDtypeStruct((T, H), jnp.float32),
    )(gate, g, g)


@jax.jit
def kernel(x, Wg, W1, b1, W2, b2):
    x_flat = x.reshape(T, H)
    dst_col, gate, te_col, loss, xb = _router(x_flat, Wg.T)
    dst_blk = dst_col.reshape(P // CH, CH)
    te = te_col.reshape(NTILES)
    xv = jax.lax.bitcast_convert_type(xb.reshape(T, HP, 2), jnp.float32)
    xs = _sc_scatter(xv, dst_blk)
    xs_b = jax.lax.bitcast_convert_type(xs, jnp.bfloat16).reshape(NROWS, H)
    ys = _ffn_grouped(te, xs_b, W1, b1, W2, b2)
    ys_v = jax.lax.bitcast_convert_type(ys.reshape(NROWS, HP, 2), jnp.float32)
    g = _sc_gather(ys_v, dst_blk)
    g_b = jax.lax.bitcast_convert_type(g, jnp.bfloat16).reshape(P, H)
    out = _combine(gate, g_b)
    return out.reshape(B, S, H), loss[0, 0]


# speedup vs baseline: 3.4158x; 3.4158x over previous
import functools

import jax
import jax.numpy as jnp
from jax.experimental import pallas as pl
from jax.experimental.pallas import tpu as pltpu
from jax.experimental.pallas import tpu_sc as plsc

B, S, H = 1, 2048, 1024
E, K, FF = 8, 2, 2048
T = B * S
P = K * T

BLK = 256
NTILES = P // BLK + E
NROWS = NTILES * BLK
MAXT = T // BLK
NROWSD = NROWS + BLK

CT = 512
HP = H // 2

NC, NS = 2, 16
NW = NC * NS
RPW = P // NW
CH = 32
NCH = RPW // CH


def _router_kernel(x_ref, wgt_ref, dst_ref, gate_ref, tm_ref, loss_ref):
    x = x_ref[...]
    wgt = wgt_ref[...]
    logits = jnp.dot(x, wgt, preferred_element_type=jnp.float32)
    iota_e = jax.lax.broadcasted_iota(jnp.int32, (T, E), 1)
    m1 = jnp.max(logits, axis=1, keepdims=True)
    a1 = jnp.min(jnp.where(logits == m1, iota_e, E), axis=1,
                 keepdims=True)
    masked = jnp.where(iota_e == a1, -jnp.inf, logits)
    m2 = jnp.max(masked, axis=1, keepdims=True)
    a2 = jnp.min(jnp.where(masked == m2, iota_e, E), axis=1,
                 keepdims=True)
    w0 = jax.nn.sigmoid(m1 - m2)
    gate_ref[:, 0:1] = w0
    gate_ref[:, 1:2] = 1.0 - w0

    a_pair = jnp.concatenate([a1, a2], axis=0)
    iota_pe = jax.lax.broadcasted_iota(jnp.int32, (P, E), 1)
    oh = (iota_pe == a_pair).astype(jnp.float32)

    c = oh
    s = 1
    while s < P:
        c = c + jnp.concatenate(
            [jnp.zeros((s, E), jnp.float32), c[:P - s, :]], axis=0)
        s *= 2
    ranks = c - oh

    counts = jnp.sum(oh, axis=0, keepdims=True)
    tiles_e = jnp.floor((counts + (BLK - 1)) * (1.0 / BLK))
    padded = tiles_e * BLK
    incl = padded
    cumt = tiles_e
    for s in (1, 2, 4):
        z = jnp.zeros((1, s), jnp.float32)
        incl = incl + jnp.concatenate([z, incl[:, :E - s]], axis=1)
        cumt = cumt + jnp.concatenate([z, cumt[:, :E - s]], axis=1)
    off = incl - padded

    rank_sel = jnp.sum(oh * ranks, axis=1, keepdims=True)
    off_sel = jnp.sum(oh * off, axis=1, keepdims=True)
    dst_ref[...] = (rank_sel + off_sel).astype(jnp.int32)

    iota_t = jax.lax.broadcasted_iota(jnp.int32, (NTILES, E), 0)
    tm_ref[...] = jnp.sum((iota_t >= cumt).astype(jnp.int32), axis=1,
                          keepdims=True)

    load = counts * (1.0 / T)
    loss_ref[...] = (0.01 * jnp.sum((load - 1.0 / E) ** 2)).reshape(1, 1)


def _router(x_flat, wgt):
    return pl.pallas_call(
        _router_kernel,
        out_shape=(
            jax.ShapeDtypeStruct((P, 1), jnp.int32),
            jax.ShapeDtypeStruct((T, K), jnp.float32),
            jax.ShapeDtypeStruct((NTILES, 1), jnp.int32),
            jax.ShapeDtypeStruct((1, 1), jnp.float32),
        ),
    )(x_flat, wgt)


def _vmesh():
    return plsc.VectorSubcoreMesh(core_axis_name="core",
                                  subcore_axis_name="subcore")


def _sc_scatter(x_flat, dst_blk):
    @functools.partial(
        pl.kernel, mesh=_vmesh(),
        out_type=jax.ShapeDtypeStruct((NROWS, H), jnp.float32),
        scratch_types=[
            pltpu.VMEM((NCH, CH), jnp.int32),
            pltpu.VMEM((CH, H), jnp.float32),
            pltpu.VMEM((CH, H), jnp.float32),
            pltpu.SemaphoreType.DMA,
            pltpu.SemaphoreType.DMA,
            pltpu.SemaphoreType.DMA,
            pltpu.SemaphoreType.DMA,
        ])
    def scatter_kernel(x_hbm, i_hbm, o_hbm, idx_v, ra, rb, sxa, sxb, ssa, ssb):
        wid = (jax.lax.axis_index("subcore") * NC
               + jax.lax.axis_index("core"))
        pltpu.sync_copy(i_hbm.at[pl.ds(wid * NCH, NCH)], idx_v)

        def src(q):
            return jax.lax.rem(wid * RPW + q * CH, T)

        bufs = (ra, rb)
        xsem = (sxa, sxb)
        ssem = (ssa, ssb)
        loads = {}
        stores = {}
        for q in (0, 1):
            loads[q] = pltpu.async_copy(
                x_hbm.at[pl.ds(src(q), CH)], bufs[q % 2], xsem[q % 2])
        for q in range(NCH):
            loads[q].wait()
            stores[q] = pltpu.async_copy(
                bufs[q % 2], o_hbm.at[idx_v.at[q]], ssem[q % 2])
            nq = q + 2
            if nq < NCH:
                stores[q].wait()
                loads[nq] = pltpu.async_copy(
                    x_hbm.at[pl.ds(src(nq), CH)], bufs[nq % 2], xsem[nq % 2])
        stores[NCH - 2].wait()
        stores[NCH - 1].wait()

    return scatter_kernel(x_flat, dst_blk)


def _sc_gather(ys, dst_blk):
    @functools.partial(
        pl.kernel, mesh=_vmesh(),
        out_type=jax.ShapeDtypeStruct((P, H), jnp.float32),
        scratch_types=[
            pltpu.VMEM((NCH, CH), jnp.int32),
            pltpu.VMEM((CH, H), jnp.float32),
            pltpu.VMEM((CH, H), jnp.float32),
            pltpu.SemaphoreType.DMA,
            pltpu.SemaphoreType.DMA,
            pltpu.SemaphoreType.DMA,
            pltpu.SemaphoreType.DMA,
        ])
    def gather_kernel(ys_hbm, i_hbm, o_hbm, idx_v, ra, rb, sga, sgb, swa, swb):
        wid = (jax.lax.axis_index("subcore") * NC
               + jax.lax.axis_index("core"))
        base = wid * RPW
        pltpu.sync_copy(i_hbm.at[pl.ds(wid * NCH, NCH)], idx_v)

        bufs = (ra, rb)
        gsem = (sga, sgb)
        wsem = (swa, swb)
        gets = {}
        puts = {}
        for q in (0, 1):
            gets[q] = pltpu.async_copy(
                ys_hbm.at[idx_v.at[q]], bufs[q % 2], gsem[q % 2])
        for q in range(NCH):
            gets[q].wait()
            puts[q] = pltpu.async_copy(
                bufs[q % 2], o_hbm.at[pl.ds(base + q * CH, CH)], wsem[q % 2])
            nq = q + 2
            if nq < NCH:
                puts[q].wait()
                gets[nq] = pltpu.async_copy(
                    ys_hbm.at[idx_v.at[nq]], bufs[nq % 2], gsem[nq % 2])
        puts[NCH - 2].wait()
        puts[NCH - 1].wait()

    return gather_kernel(ys, dst_blk)


def _ffn_kernel(te_ref, xs_ref, w1_ref, b1_ref, w2_ref, b2_ref, ys_ref):
    j = pl.program_id(0)

    @pl.when(te_ref[j] < E)
    def _():
        x = xs_ref[...]
        h = jnp.dot(x.astype(jnp.bfloat16), w1_ref[0].astype(jnp.bfloat16),
                    preferred_element_type=jnp.float32) + b1_ref[0]
        h = jax.nn.gelu(h)
        y = jnp.dot(h.astype(jnp.bfloat16), w2_ref[0].astype(jnp.bfloat16),
                    preferred_element_type=jnp.float32) + b2_ref[0]
        ys_ref[...] = y


def _ffn_grouped(te, xs, W1, b1, W2, b2):
    def emap(j, te_ref):
        return (jnp.minimum(te_ref[j], E - 1),)

    grid_spec = pltpu.PrefetchScalarGridSpec(
        num_scalar_prefetch=1,
        grid=(NTILES,),
        in_specs=[
            pl.BlockSpec((BLK, H), lambda j, te_ref: (j, 0)),
            pl.BlockSpec((1, H, FF), lambda j, te_ref: (*emap(j, te_ref), 0, 0)),
            pl.BlockSpec((1, 1, FF), lambda j, te_ref: (*emap(j, te_ref), 0, 0)),
            pl.BlockSpec((1, FF, H), lambda j, te_ref: (*emap(j, te_ref), 0, 0)),
            pl.BlockSpec((1, 1, H), lambda j, te_ref: (*emap(j, te_ref), 0, 0)),
        ],
        out_specs=pl.BlockSpec((BLK, H), lambda j, te_ref: (j, 0)),
    )
    return pl.pallas_call(
        _ffn_kernel,
        grid_spec=grid_spec,
        out_shape=jax.ShapeDtypeStruct((NROWS, H), jnp.float32),
        compiler_params=pltpu.CompilerParams(
            dimension_semantics=("arbitrary",),
        ),
    )(te, xs, W1, b1.reshape(E, 1, FF), W2, b2.reshape(E, 1, H))


def _combine_kernel(gate_ref, g0_ref, g1_ref, out_ref):
    w0 = gate_ref[:, 0:1]
    w1 = gate_ref[:, 1:2]
    out_ref[...] = w0 * g0_ref[...] + w1 * g1_ref[...]


def _combine(gate, g):
    return pl.pallas_call(
        _combine_kernel,
        grid=(T // CT,),
        in_specs=[
            pl.BlockSpec((CT, K), lambda c: (c, 0)),
            pl.BlockSpec((CT, H), lambda c: (c, 0)),
            pl.BlockSpec((CT, H), lambda c: (T // CT + c, 0)),
        ],
        out_specs=pl.BlockSpec((CT, H), lambda c: (c, 0)),
        out_shape=jax.ShapeDtypeStruct((T, H), jnp.float32),
    )(gate, g, g)


@jax.jit
def kernel(x, Wg, W1, b1, W2, b2):
    x_flat = x.reshape(T, H)
    dst_col, gate, te_col, loss = _router(x_flat, Wg.T)
    dst_blk = dst_col.reshape(P // CH, CH)
    te = te_col.reshape(NTILES)
    xs = _sc_scatter(x_flat, dst_blk)
    ys = _ffn_grouped(te, xs, W1, b1, W2, b2)
    g = _sc_gather(ys, dst_blk)
    out = _combine(gate, g)
    return out.reshape(B, S, H), loss[0, 0]
